# baseline (speedup 1.0000x reference)
import jax
import jax.numpy as jnp
from jax import lax
from jax.experimental import pallas as pl
from jax.experimental.pallas import tpu as pltpu

_K = 0.05
_EPS = 0.3


def _body(x_ref, y_ref, d_ref, z_ref, out_ref, acc_ref):
    step = pl.program_id(0)
    nsteps = pl.num_programs(0)

    @pl.when(step == 0)
    def _init():
        acc_ref[0] = 0.0
        acc_ref[1] = 0.0
        acc_ref[2] = -jnp.inf

    xb = x_ref[...]
    yb = y_ref[...]
    cols = lax.broadcasted_iota(jnp.int32, xb.shape, 1)
    onehot = cols == yb
    target = jnp.sum(jnp.where(onehot, xb, 0.0), axis=1)
    rest_max = jnp.max(jnp.where(onehot, -jnp.inf, xb), axis=1)
    relu_sum = jnp.sum(jnp.maximum(target - rest_max + _K, 0.0))

    zb = z_ref[...]
    db = d_ref[...]

    acc_ref[0] += relu_sum
    acc_ref[1] += jnp.sum(zb)
    acc_ref[2] = jnp.maximum(acc_ref[2], jnp.max(db))

    @pl.when(step == nsteps - 1)
    def _fini():
        b = jnp.float32(nsteps) * jnp.float32(xb.shape[0])
        out_ref[0, 0] = acc_ref[0] / b + (acc_ref[1] / b) * (_EPS + acc_ref[2])


def kernel(x, delta, y, z):
    B, C = x.shape
    BR = 512
    grid = B // BR

    y2 = y.reshape(B, 1).astype(jnp.int32)
    z2 = z.reshape(B, 1)

    out = pl.pallas_call(
        _body,
        grid=(grid,),
        in_specs=[
            pl.BlockSpec((BR, C), lambda i: (i, 0)),
            pl.BlockSpec((BR, 1), lambda i: (i, 0)),
            pl.BlockSpec((BR, delta.shape[1]), lambda i: (i, 0)),
            pl.BlockSpec((BR, 1), lambda i: (i, 0)),
        ],
        out_specs=pl.BlockSpec(
            (1, 1), lambda i: (0, 0), memory_space=pltpu.SMEM
        ),
        out_shape=jax.ShapeDtypeStruct((1, 1), jnp.float32),
        scratch_shapes=[pltpu.SMEM((3,), jnp.float32)],
    )(x, y2, delta, z2)
    return out[0, 0]


# speedup vs baseline: 1.0501x; 1.0501x over previous
import jax
import jax.numpy as jnp
from jax import lax
from jax.experimental import pallas as pl
from jax.experimental.pallas import tpu as pltpu

_K = 0.05
_EPS = 0.3


def _body(x_ref, y_ref, d_ref, z_ref, out_ref, acc_ref):
    step = pl.program_id(0)
    nsteps = pl.num_programs(0)

    @pl.when(step == 0)
    def _init():
        acc_ref[0] = 0.0
        acc_ref[1] = 0.0
        acc_ref[2] = -jnp.inf

    xb = x_ref[...]
    yb = y_ref[...]
    cols = lax.broadcasted_iota(jnp.int32, xb.shape, 1)
    onehot = cols == yb
    target = jnp.sum(jnp.where(onehot, xb, 0.0), axis=1)
    rest_max = jnp.max(jnp.where(onehot, -jnp.inf, xb), axis=1)
    relu_sum = jnp.sum(jnp.maximum(target - rest_max + _K, 0.0))

    zb = z_ref[...]
    db = d_ref[...]

    acc_ref[0] += relu_sum
    acc_ref[1] += jnp.sum(zb)
    acc_ref[2] = jnp.maximum(acc_ref[2], jnp.max(db))

    @pl.when(step == nsteps - 1)
    def _fini():
        b = jnp.float32(nsteps) * jnp.float32(xb.shape[0])
        out_ref[0, 0] = acc_ref[0] / b + (acc_ref[1] / b) * (_EPS + acc_ref[2])


def kernel(x, delta, y, z):
    B, C = x.shape
    BR = 1024
    grid = B // BR

    y2 = y.reshape(B, 1).astype(jnp.int32)
    z2 = z.reshape(B, 1)

    out = pl.pallas_call(
        _body,
        grid=(grid,),
        in_specs=[
            pl.BlockSpec((BR, C), lambda i: (i, 0)),
            pl.BlockSpec((BR, 1), lambda i: (i, 0)),
            pl.BlockSpec((BR, delta.shape[1]), lambda i: (i, 0)),
            pl.BlockSpec((BR, 1), lambda i: (i, 0)),
        ],
        out_specs=pl.BlockSpec(
            (1, 1), lambda i: (0, 0), memory_space=pltpu.SMEM
        ),
        out_shape=jax.ShapeDtypeStruct((1, 1), jnp.float32),
        scratch_shapes=[pltpu.SMEM((3,), jnp.float32)],
    )(x, y2, delta, z2)
    return out[0, 0]


# speedup vs baseline: 1.1491x; 1.0942x over previous
import jax
import jax.numpy as jnp
from jax import lax
from jax.experimental import pallas as pl
from jax.experimental.pallas import tpu as pltpu

_K = 0.05
_EPS = 0.3


def _body(x_ref, y_ref, d_ref, z_ref, out_ref, acc_ref):
    step = pl.program_id(0)
    nsteps = pl.num_programs(0)

    @pl.when(step == 0)
    def _init():
        acc_ref[0] = 0.0
        acc_ref[1] = 0.0
        acc_ref[2] = -jnp.inf

    xb = x_ref[...]
    yb = y_ref[...].reshape(xb.shape[0], 1)
    cols = lax.broadcasted_iota(jnp.int32, xb.shape, 1)
    onehot = cols == yb
    target = jnp.sum(jnp.where(onehot, xb, 0.0), axis=1)
    rest_max = jnp.max(jnp.where(onehot, -jnp.inf, xb), axis=1)
    relu_sum = jnp.sum(jnp.maximum(target - rest_max + _K, 0.0))

    zb = z_ref[...]
    db = d_ref[...]

    acc_ref[0] += relu_sum
    acc_ref[1] += jnp.sum(zb)
    acc_ref[2] = jnp.maximum(acc_ref[2], jnp.max(db))

    @pl.when(step == nsteps - 1)
    def _fini():
        b = jnp.float32(nsteps) * jnp.float32(xb.shape[0])
        out_ref[0, 0] = acc_ref[0] / b + (acc_ref[1] / b) * (_EPS + acc_ref[2])


def kernel(x, delta, y, z):
    B, C = x.shape
    BR = 1024
    grid = B // BR

    out = pl.pallas_call(
        _body,
        grid=(grid,),
        in_specs=[
            pl.BlockSpec((BR, C), lambda i: (i, 0)),
            pl.BlockSpec((BR,), lambda i: (i,)),
            pl.BlockSpec((BR, delta.shape[1]), lambda i: (i, 0)),
            pl.BlockSpec((BR,), lambda i: (i,)),
        ],
        out_specs=pl.BlockSpec(
            (1, 1), lambda i: (0, 0), memory_space=pltpu.SMEM
        ),
        out_shape=jax.ShapeDtypeStruct((1, 1), jnp.float32),
        scratch_shapes=[pltpu.SMEM((3,), jnp.float32)],
    )(x, y.astype(jnp.int32), delta, z)
    return out[0, 0]


# speedup vs baseline: 4.1920x; 3.6483x over previous
import jax
import jax.numpy as jnp
from jax import lax
from jax.experimental import pallas as pl
from jax.experimental.pallas import tpu as pltpu

_K = 0.05
_EPS = 0.3


def _body(x_ref, y_ref, d_ref, z_ref, out_ref, acc_ref):
    step = pl.program_id(0)
    nsteps = pl.num_programs(0)

    @pl.when(step == 0)
    def _init():
        acc_ref[0] = 0.0
        acc_ref[1] = 0.0
        acc_ref[2] = -jnp.inf

    xb = x_ref[...]
    yb = y_ref[...][None, :]
    rows = lax.broadcasted_iota(jnp.int32, xb.shape, 0)
    onehot = rows == yb
    target = jnp.sum(jnp.where(onehot, xb, 0.0), axis=0)
    rest_max = jnp.max(jnp.where(onehot, -jnp.inf, xb), axis=0)
    relu_sum = jnp.sum(jnp.maximum(target - rest_max + _K, 0.0))

    zb = z_ref[...]
    db = d_ref[...]

    acc_ref[0] += relu_sum
    acc_ref[1] += jnp.sum(zb)
    acc_ref[2] = jnp.maximum(acc_ref[2], jnp.max(db))

    @pl.when(step == nsteps - 1)
    def _fini():
        b = jnp.float32(nsteps) * jnp.float32(xb.shape[1])
        out_ref[0, 0] = acc_ref[0] / b + (acc_ref[1] / b) * (_EPS + acc_ref[2])


def kernel(x, delta, y, z):
    B, C = x.shape
    D = delta.shape[1]
    BB = 1024
    grid = B // BB

    xt = x.T
    dt = delta.T

    out = pl.pallas_call(
        _body,
        grid=(grid,),
        in_specs=[
            pl.BlockSpec((C, BB), lambda i: (0, i)),
            pl.BlockSpec((BB,), lambda i: (i,)),
            pl.BlockSpec((D, BB), lambda i: (0, i)),
            pl.BlockSpec((BB,), lambda i: (i,)),
        ],
        out_specs=pl.BlockSpec(
            (1, 1), lambda i: (0, 0), memory_space=pltpu.SMEM
        ),
        out_shape=jax.ShapeDtypeStruct((1, 1), jnp.float32),
        scratch_shapes=[pltpu.SMEM((3,), jnp.float32)],
    )(xt, y.astype(jnp.int32), dt, z)
    return out[0, 0]


# speedup vs baseline: 4.4536x; 1.0624x over previous
import jax
import jax.numpy as jnp
from jax import lax
from jax.experimental import pallas as pl
from jax.experimental.pallas import tpu as pltpu

_K = 0.05
_EPS = 0.3


def _body(x_ref, y_ref, d_ref, z_ref, out_ref, acc_ref):
    step = pl.program_id(0)
    nsteps = pl.num_programs(0)

    @pl.when(step == 0)
    def _init():
        acc_ref[0] = 0.0
        acc_ref[1] = 0.0
        acc_ref[2] = -jnp.inf

    xb = x_ref[...]
    yb = y_ref[...][None, :]
    rows = lax.broadcasted_iota(jnp.int32, xb.shape, 0)
    onehot = rows == yb
    target = jnp.sum(jnp.where(onehot, xb, 0.0), axis=0)
    rest_max = jnp.max(jnp.where(onehot, -jnp.inf, xb), axis=0)
    relu_sum = jnp.sum(jnp.maximum(target - rest_max + _K, 0.0))

    zb = z_ref[...]
    db = d_ref[...]

    acc_ref[0] += relu_sum
    acc_ref[1] += jnp.sum(zb)
    acc_ref[2] = jnp.maximum(acc_ref[2], jnp.max(db))

    @pl.when(step == nsteps - 1)
    def _fini():
        b = jnp.float32(nsteps) * jnp.float32(xb.shape[1])
        out_ref[0, 0] = acc_ref[0] / b + (acc_ref[1] / b) * (_EPS + acc_ref[2])


def kernel(x, delta, y, z):
    B, C = x.shape
    D = delta.shape[1]
    BB = 2048
    grid = B // BB

    xt = x.T
    dt = delta.T

    out = pl.pallas_call(
        _body,
        grid=(grid,),
        in_specs=[
            pl.BlockSpec((C, BB), lambda i: (0, i)),
            pl.BlockSpec((BB,), lambda i: (i,)),
            pl.BlockSpec((D, BB), lambda i: (0, i)),
            pl.BlockSpec((BB,), lambda i: (i,)),
        ],
        out_specs=pl.BlockSpec(
            (1, 1), lambda i: (0, 0), memory_space=pltpu.SMEM
        ),
        out_shape=jax.ShapeDtypeStruct((1, 1), jnp.float32),
        scratch_shapes=[pltpu.SMEM((3,), jnp.float32)],
    )(xt, y.astype(jnp.int32), dt, z)
    return out[0, 0]
